# baseline (speedup 1.0000x reference)
import functools

import jax
import jax.numpy as jnp
from jax import lax
from jax.experimental import pallas as pl
from jax.experimental.pallas import tpu as pltpu
from jax.experimental.pallas import tpu_sc as plsc

_LANES = 16
_NUM_WORKERS = 32
_NSC = 16
_CHUNK = 3200
_NSUB = 2


@functools.lru_cache(maxsize=None)
def _build_impl(E: int, V: int):
    V_pad = -(V // -128) * 128
    E_pad = -(E // -128) * 128
    per_w = _NSUB * _CHUNK
    last_base = E_pad - per_w
    step = -(last_base // -(128 * (_NUM_WORKERS - 1))) * 128
    assert step <= per_w and (_NUM_WORKERS - 2) * step + per_w >= last_base
    stripe = V_pad // _NSC
    assert stripe % 8 == 0

    mesh = plsc.VectorSubcoreMesh(core_axis_name="c", subcore_axis_name="s")

    in_block = lambda dt: pltpu.VMEM((3 * _CHUNK,), dt)

    @functools.partial(
        pl.kernel,
        out_type=jax.ShapeDtypeStruct((2 * E_pad,), jnp.float32),
        mesh=mesh,
        compiler_params=pltpu.CompilerParams(needs_layout_passes=False),
        scratch_types=[
            pltpu.VMEM_SHARED((V_pad,), jnp.float32),
            [in_block(jnp.int32) for _ in range(2)],
            [in_block(jnp.float32) for _ in range(2)],
            [in_block(jnp.float32) for _ in range(2)],
            [in_block(jnp.float32) for _ in range(2)],
            [pltpu.VMEM((2 * _CHUNK,), jnp.float32) for _ in range(2)],
            pltpu.VMEM((stripe,), jnp.float32),
            [pltpu.SemaphoreType.DMA for _ in range(2)],
            [pltpu.SemaphoreType.DMA for _ in range(2)],
            [pltpu.SemaphoreType.DMA for _ in range(2)],
        ],
    )
    def _impl(field_hbm, e0_hbm, e1_hbm, e2_hbm, x0_hbm, x1_hbm, x2_hbm,
              y0_hbm, y1_hbm, y2_hbm, out_hbm,
              field_sh, el_v, wx_v, wy_v, f_v, out_v, stage_v,
              sem_in, sem_fg, sem_out):
        cid = lax.axis_index("c")
        sid = lax.axis_index("s")
        wid = (sid * 2 + cid).astype(jnp.int32)
        base_w = jnp.minimum(wid * step, last_base)

        def issue_in(s, b):
            sc_base = base_w + s * _CHUNK
            hs = []
            for cols, arr_v in (((e0_hbm, e1_hbm, e2_hbm), el_v),
                                ((x0_hbm, x1_hbm, x2_hbm), wx_v),
                                ((y0_hbm, y1_hbm, y2_hbm), wy_v)):
                for k in range(3):
                    hs.append(pltpu.async_copy(
                        cols[k].at[pl.ds(sc_base, _CHUNK)],
                        arr_v[b].at[pl.ds(k * _CHUNK, _CHUNK)],
                        sem_in[b]))
            return hs

        pend_in = {0: issue_in(0, 0)}
        if _NSUB > 1:
            pend_in[1] = issue_in(1, 1)

        s_lo = sid * stripe
        pltpu.sync_copy(field_hbm.at[pl.ds(s_lo, stripe)], stage_v)
        pltpu.sync_copy(stage_v, field_sh.at[pl.ds(s_lo, stripe)])
        plsc.subcore_barrier()

        def fixup_tail(s, b):
            sc_base = base_w + s * _CHUNK
            @pl.when(sc_base + _CHUNK > E)
            def _():
                hi = jnp.full((_LANES,), V - 1, jnp.int32)
                lo = jnp.zeros((_LANES,), jnp.int32)
                for k in range(3):
                    for u in range(8):
                        o = k * _CHUNK + _CHUNK - 128 + u * _LANES
                        el_v[b][pl.ds(o, _LANES)] = jnp.minimum(
                            jnp.maximum(el_v[b][pl.ds(o, _LANES)], lo), hi)

        def issue_gather(s, b):
            hs = []
            for k in range(3):
                hs.append(pltpu.async_copy(
                    field_sh.at[el_v[b].at[pl.ds(k * _CHUNK, _CHUNK)]],
                    f_v[b].at[pl.ds(k * _CHUNK, _CHUNK)],
                    sem_fg[b]))
            return hs

        for h in pend_in.pop(0):
            h.wait()
        fixup_tail(0, 0)
        pend_fg = {0: issue_gather(0, 0)}
        pend_out = {}

        for s in range(_NSUB):
            b = s % 2
            if s + 1 < _NSUB:
                nb = (s + 1) % 2
                for h in pend_in.pop(s + 1):
                    h.wait()
                fixup_tail(s + 1, nb)
                pend_fg[s + 1] = issue_gather(s + 1, nb)
            for h in pend_fg.pop(s):
                h.wait()
            if s >= 2:
                pend_out.pop(s - 2).wait()

            def body(g, _, b=b):
                for u in range(8):
                    o = g * 128 + u * _LANES
                    f0 = f_v[b][pl.ds(o, _LANES)]
                    f1 = f_v[b][pl.ds(_CHUNK + o, _LANES)]
                    f2 = f_v[b][pl.ds(2 * _CHUNK + o, _LANES)]
                    gx = (wx_v[b][pl.ds(o, _LANES)] * f0
                          + wx_v[b][pl.ds(_CHUNK + o, _LANES)] * f1
                          + wx_v[b][pl.ds(2 * _CHUNK + o, _LANES)] * f2)
                    gy = (wy_v[b][pl.ds(o, _LANES)] * f0
                          + wy_v[b][pl.ds(_CHUNK + o, _LANES)] * f1
                          + wy_v[b][pl.ds(2 * _CHUNK + o, _LANES)] * f2)
                    d = g * 256 + u * _LANES
                    out_v[b][pl.ds(d, _LANES)] = gx
                    out_v[b][pl.ds(d + 128, _LANES)] = gy
                return 0

            lax.fori_loop(0, _CHUNK // 128, body, 0)

            sc_base = base_w + s * _CHUNK
            pend_out[s] = pltpu.async_copy(
                out_v[b], out_hbm.at[pl.ds(2 * sc_base, 2 * _CHUNK)], sem_out[b])
            if s + 2 < _NSUB:
                pend_in[s + 2] = issue_in(s + 2, b)

        for s, h in sorted(pend_out.items()):
            h.wait()

    return _impl


def kernel(field, wx, wy, elements):
    V = field.shape[0]
    E = elements.shape[0]
    E_pad = -(E // -128) * 128
    el = elements.astype(jnp.int32)
    impl = _build_impl(E, V)
    out = impl(field, el[:, 0], el[:, 1], el[:, 2],
               wx[:, 0], wx[:, 1], wx[:, 2],
               wy[:, 0], wy[:, 1], wy[:, 2])
    return out.reshape(E_pad // 128, 2, 128).transpose(0, 2, 1).reshape(E_pad, 2)[:E]


# speedup vs baseline: 1.5040x; 1.5040x over previous
import functools

import jax
import jax.numpy as jnp
from jax import lax
from jax.experimental import pallas as pl
from jax.experimental.pallas import tpu as pltpu
from jax.experimental.pallas import tpu_sc as plsc

_LANES = 16
_NUM_WORKERS = 32
_NSC = 16
_CHUNK = 3200
_NSUB = 2


@functools.lru_cache(maxsize=None)
def _build_impl(E: int, V: int):
    V_pad = -(V // -128) * 128
    E_pad = -(E // -128) * 128
    per_w = _NSUB * _CHUNK
    last_base = E_pad - per_w
    step = -(last_base // -(128 * (_NUM_WORKERS - 1))) * 128
    assert step <= per_w and (_NUM_WORKERS - 2) * step + per_w >= last_base
    stripe = V_pad // _NSC
    assert stripe % 8 == 0

    mesh = plsc.VectorSubcoreMesh(core_axis_name="c", subcore_axis_name="s")

    in_block = lambda dt: pltpu.VMEM((3 * _CHUNK,), dt)

    @functools.partial(
        pl.kernel,
        out_type=jax.ShapeDtypeStruct((2 * E_pad,), jnp.float32),
        mesh=mesh,
        compiler_params=pltpu.CompilerParams(needs_layout_passes=False),
        scratch_types=[
            pltpu.VMEM_SHARED((V_pad,), jnp.float32),
            [in_block(jnp.int32) for _ in range(2)],
            [in_block(jnp.float32) for _ in range(2)],
            [in_block(jnp.float32) for _ in range(2)],
            [in_block(jnp.float32) for _ in range(2)],
            [pltpu.VMEM((2 * _CHUNK,), jnp.float32) for _ in range(2)],
            pltpu.VMEM((stripe,), jnp.float32),
            [pltpu.SemaphoreType.DMA for _ in range(2)],
            [pltpu.SemaphoreType.DMA for _ in range(2)],
            [pltpu.SemaphoreType.DMA for _ in range(2)],
        ],
    )
    def _impl(field_hbm, el_hbm, wx_hbm, wy_hbm, out_hbm,
              field_sh, el_v, wx_v, wy_v, f_v, out_v, stage_v,
              sem_in, sem_fg, sem_out):
        cid = lax.axis_index("c")
        sid = lax.axis_index("s")
        wid = (sid * 2 + cid).astype(jnp.int32)
        base_w = jnp.minimum(wid * step, last_base)

        def issue_in(s, b):
            sc_base = base_w + s * _CHUNK
            hs = []
            for arr_hbm, arr_v in ((el_hbm, el_v), (wx_hbm, wx_v), (wy_hbm, wy_v)):
                for k in range(3):
                    hs.append(pltpu.async_copy(
                        arr_hbm.at[pl.ds(k * E + sc_base, _CHUNK)],
                        arr_v[b].at[pl.ds(k * _CHUNK, _CHUNK)],
                        sem_in[b]))
            return hs

        pend_in = {0: issue_in(0, 0)}
        if _NSUB > 1:
            pend_in[1] = issue_in(1, 1)

        s_lo = sid * stripe
        pltpu.sync_copy(field_hbm.at[pl.ds(s_lo, stripe)], stage_v)
        pltpu.sync_copy(stage_v, field_sh.at[pl.ds(s_lo, stripe)])
        plsc.subcore_barrier()

        def fixup_tail(s, b):
            sc_base = base_w + s * _CHUNK
            @pl.when(sc_base + _CHUNK > E)
            def _():
                hi = jnp.full((_LANES,), V - 1, jnp.int32)
                lo = jnp.zeros((_LANES,), jnp.int32)
                for k in range(3):
                    for u in range(8):
                        o = k * _CHUNK + _CHUNK - 128 + u * _LANES
                        el_v[b][pl.ds(o, _LANES)] = jnp.minimum(
                            jnp.maximum(el_v[b][pl.ds(o, _LANES)], lo), hi)

        def issue_gather(s, b):
            hs = []
            for k in range(3):
                hs.append(pltpu.async_copy(
                    field_sh.at[el_v[b].at[pl.ds(k * _CHUNK, _CHUNK)]],
                    f_v[b].at[pl.ds(k * _CHUNK, _CHUNK)],
                    sem_fg[b]))
            return hs

        for h in pend_in.pop(0):
            h.wait()
        fixup_tail(0, 0)
        pend_fg = {0: issue_gather(0, 0)}
        pend_out = {}

        for s in range(_NSUB):
            b = s % 2
            if s + 1 < _NSUB:
                nb = (s + 1) % 2
                for h in pend_in.pop(s + 1):
                    h.wait()
                fixup_tail(s + 1, nb)
                pend_fg[s + 1] = issue_gather(s + 1, nb)
            for h in pend_fg.pop(s):
                h.wait()
            if s >= 2:
                pend_out.pop(s - 2).wait()

            def body(g, _, b=b):
                for u in range(8):
                    o = g * 128 + u * _LANES
                    f0 = f_v[b][pl.ds(o, _LANES)]
                    f1 = f_v[b][pl.ds(_CHUNK + o, _LANES)]
                    f2 = f_v[b][pl.ds(2 * _CHUNK + o, _LANES)]
                    gx = (wx_v[b][pl.ds(o, _LANES)] * f0
                          + wx_v[b][pl.ds(_CHUNK + o, _LANES)] * f1
                          + wx_v[b][pl.ds(2 * _CHUNK + o, _LANES)] * f2)
                    gy = (wy_v[b][pl.ds(o, _LANES)] * f0
                          + wy_v[b][pl.ds(_CHUNK + o, _LANES)] * f1
                          + wy_v[b][pl.ds(2 * _CHUNK + o, _LANES)] * f2)
                    d = g * 256 + u * _LANES
                    out_v[b][pl.ds(d, _LANES)] = gx
                    out_v[b][pl.ds(d + 128, _LANES)] = gy
                return 0

            lax.fori_loop(0, _CHUNK // 128, body, 0)

            sc_base = base_w + s * _CHUNK
            pend_out[s] = pltpu.async_copy(
                out_v[b], out_hbm.at[pl.ds(2 * sc_base, 2 * _CHUNK)], sem_out[b])
            if s + 2 < _NSUB:
                pend_in[s + 2] = issue_in(s + 2, b)

        for s, h in sorted(pend_out.items()):
            h.wait()

    return _impl


def kernel(field, wx, wy, elements):
    V = field.shape[0]
    E = elements.shape[0]
    E_pad = -(E // -128) * 128
    elT = elements.astype(jnp.int32).T.reshape(-1)
    wxT = wx.T.reshape(-1)
    wyT = wy.T.reshape(-1)
    impl = _build_impl(E, V)
    out = impl(field, elT, wxT, wyT)
    return out.reshape(E_pad // 128, 2, 128).transpose(0, 2, 1).reshape(E_pad, 2)[:E]
